# baseline (device time: 41872 ns/iter reference)
import jax
import jax.numpy as jnp
from jax import lax
from jax.experimental import pallas as pl
from jax.experimental.pallas import tpu as pltpu

N_DEV = 8
MASKS = (1, 3, 4)
NG = 8
W = (128,) * NG
COLS = tuple(128 * g for g in range(NG))
SIZES = (1024, 512, 256)
SZA = (256, 512, 1024)
SB_BASE = (0, 1024, 1536)


def _aligned(x, m):
    return pl.multiple_of(x, m)


def kernel(t):
    m, n = t.shape
    assert (m, n) == (2048, 1024)
    bf16 = jnp.bfloat16

    def body(x_ref, out_ref, acc_ref, sb, c1, c2, c3, rs_send_sems, ag_send_sems, rs_sems, ag_sems):
        my = lax.axis_index("i")
        comms = [c1, c2, c3]
        bit0, bit1, bit2 = my & 1, (my >> 1) & 1, (my >> 2) & 1
        beta_of = {1: bit0 ^ bit1, 3: bit1, 4: bit2}

        barrier = pltpu.get_barrier_semaphore()
        for mask in MASKS:
            pl.semaphore_signal(
                barrier, inc=1,
                device_id=(my ^ mask,), device_id_type=pl.DeviceIdType.MESH,
            )
        pl.semaphore_wait(barrier, 3)

        offs = [jnp.int32(0)] * NG

        def rs_send(g, s):
            mask = MASKS[(g + s) % 3]
            half = SIZES[s]
            lower = beta_of[mask] == 0
            cg, w = COLS[g], W[g]
            send_lo = _aligned(
                jnp.where(lower, jnp.int32(half), jnp.int32(0)), half
            )
            if s == 0:
                sb[pl.ds(0, half), pl.ds(cg, w)] = x_ref[
                    pl.ds(send_lo, half), pl.ds(cg, w)
                ].astype(bf16)
                src = sb.at[pl.ds(0, half), pl.ds(cg, w)]
            else:
                src = acc_ref.at[pl.ds(send_lo, half), pl.ds(cg, w)]
            rdma = pltpu.make_async_remote_copy(
                src_ref=src,
                dst_ref=comms[s].at[:, pl.ds(cg, w)],
                send_sem=rs_send_sems.at[g, s],
                recv_sem=rs_sems.at[g, s],
                device_id=(my ^ mask,),
                device_id_type=pl.DeviceIdType.MESH,
            )
            rdma.start()
            return rdma

        def rs_accum(g, s):
            mask = MASKS[(g + s) % 3]
            half = SIZES[s]
            lower = beta_of[mask] == 0
            cg, w = COLS[g], W[g]
            keep_lo = _aligned(
                jnp.where(lower, jnp.int32(0), jnp.int32(half)), half
            )
            if s == 0:
                kept = x_ref[pl.ds(keep_lo, half), pl.ds(cg, w)]
            else:
                kept = acc_ref[pl.ds(keep_lo, half), pl.ds(cg, w)].astype(
                    jnp.float32
                )
            acc_ref[pl.ds(0, half), pl.ds(cg, w)] = (
                kept + comms[s][:, pl.ds(cg, w)].astype(jnp.float32)
            ).astype(bf16)
            offs[g] = offs[g] + jnp.where(lower, jnp.int32(0), jnp.int32(half))

        rs_d = [[None] * 3 for _ in range(NG)]
        for g in range(NG):
            rs_d[g][0] = rs_send(g, 0)
        for s in (1, 2):
            for g in range(NG):
                rs_d[g][s - 1].wait_recv()
                if s - 1 >= 1:
                    rs_d[g][s - 1].wait_send()
                rs_accum(g, s - 1)
                rs_d[g][s] = rs_send(g, s)
        ag_recv = [[None] * 3 for _ in range(NG)]
        ag_poff = [[None] * 3 for _ in range(NG)]

        ag_sd = [[None] * 3 for _ in range(NG)]

        def ag_start(g, a):
            mask = MASKS[(g + 2 - a) % 3]
            lower = beta_of[mask] == 0
            sz = SZA[a]
            cg, w = COLS[g], W[g]
            o = _aligned(offs[g], sz)
            send = pltpu.make_async_remote_copy(
                src_ref=out_ref.at[pl.ds(o, sz), pl.ds(cg, w)],
                dst_ref=out_ref.at[pl.ds(o, sz), pl.ds(cg, w)],
                send_sem=ag_send_sems.at[g, a],
                recv_sem=ag_sems.at[g, a],
                device_id=(my ^ mask,),
                device_id_type=pl.DeviceIdType.MESH,
            )
            send.start()
            ag_sd[g][a] = send
            p_off = _aligned(jnp.where(lower, o + sz, o - sz), sz)
            recv = pltpu.make_async_remote_copy(
                src_ref=out_ref.at[pl.ds(p_off, sz), pl.ds(cg, w)],
                dst_ref=out_ref.at[pl.ds(p_off, sz), pl.ds(cg, w)],
                send_sem=ag_send_sems.at[g, a],
                recv_sem=ag_sems.at[g, a],
                device_id=(my ^ mask,),
                device_id_type=pl.DeviceIdType.MESH,
            )
            ag_recv[g][a] = recv
            ag_poff[g][a] = p_off

        for g in range(NG):
            rs_d[g][2].wait_recv()
            rs_d[g][2].wait_send()
            rs_accum(g, 2)
            cg, w = COLS[g], W[g]
            s_val = acc_ref[pl.ds(0, 256), pl.ds(cg, w)].astype(jnp.float32)
            relu = jnp.maximum(s_val, 0.0)
            y = jnp.tanh(s_val) * s_val * s_val + relu * relu * relu
            out_ref[pl.ds(_aligned(offs[g], 256), 256), pl.ds(cg, w)] = (
                y.astype(bf16)
            )
            ag_start(g, 0)

        for a in (1, 2):
            for g in range(NG):
                ag_recv[g][a - 1].wait_recv()
                offs[g] = jnp.minimum(offs[g], ag_poff[g][a - 1])
                ag_start(g, a)
        for g in range(NG):
            ag_recv[g][2].wait_recv()
        for g in range(NG):
            rs_d[g][0].wait_send()
            for s in range(3):
                ag_sd[g][s].wait_send()

    return pl.pallas_call(
        body,
        out_shape=jax.ShapeDtypeStruct((m, n), bf16),
        in_specs=[pl.BlockSpec(memory_space=pltpu.VMEM)],
        out_specs=pl.BlockSpec(memory_space=pltpu.VMEM),
        scratch_shapes=[
            pltpu.VMEM((m // 2, n), bf16),
            pltpu.VMEM((m // 2, n), bf16),
            pltpu.VMEM((m // 2, n), bf16),
            pltpu.VMEM((m // 4, n), bf16),
            pltpu.VMEM((m // 8, n), bf16),
            pltpu.SemaphoreType.DMA((NG, 3)),
            pltpu.SemaphoreType.DMA((NG, 3)),
            pltpu.SemaphoreType.DMA((NG, 3)),
            pltpu.SemaphoreType.DMA((NG, 3)),
        ],
        compiler_params=pltpu.CompilerParams(collective_id=0),
    )(t)


# device time: 39689 ns/iter; 1.0550x vs baseline; 1.0550x over previous
import jax
import jax.numpy as jnp
from jax import lax
from jax.experimental import pallas as pl
from jax.experimental.pallas import tpu as pltpu

N_DEV = 8
MASKS = (1, 3, 4)
NG = 8
W = (128,) * NG
COLS = tuple(128 * g for g in range(NG))
SIZES = (1024, 512, 512)
SZA = (256, 512, 1024)
SB_BASE = (0, 1024, 1536)


def _aligned(x, m):
    return pl.multiple_of(x, m)


def kernel(t):
    m, n = t.shape
    assert (m, n) == (2048, 1024)
    bf16 = jnp.bfloat16

    def body(x_ref, out_ref, acc_ref, sb, c1, c2, c3,
             rs_send_sems, ag_send_sems, rs_sems, ag_sems):
        my = lax.axis_index("i")
        comms = [c1, c2, c3]
        bit0, bit1, bit2 = my & 1, (my >> 1) & 1, (my >> 2) & 1
        beta_of = {1: bit0 ^ bit1, 3: bit1, 4: bit2}

        barrier = pltpu.get_barrier_semaphore()
        for mask in MASKS:
            pl.semaphore_signal(
                barrier, inc=1,
                device_id=(my ^ mask,), device_id_type=pl.DeviceIdType.MESH,
            )
        pl.semaphore_wait(barrier, 3)

        offs = [jnp.int32(0)] * NG

        def rs_send(g, s):
            mask = MASKS[(g + s) % 3]
            rows = SIZES[s]
            cg, w = COLS[g], W[g]
            if s == 2:
                send_lo = 0
            else:
                lower = beta_of[mask] == 0
                send_lo = _aligned(
                    jnp.where(lower, jnp.int32(rows), jnp.int32(0)), rows
                )
            src = x_ref if s == 0 else acc_ref
            base = SB_BASE[s]
            sb[pl.ds(base, rows), pl.ds(cg, w)] = src[
                pl.ds(send_lo, rows), pl.ds(cg, w)
            ].astype(bf16)
            rdma = pltpu.make_async_remote_copy(
                src_ref=sb.at[pl.ds(base, rows), pl.ds(cg, w)],
                dst_ref=comms[s].at[:, pl.ds(cg, w)],
                send_sem=rs_send_sems.at[g, s],
                recv_sem=rs_sems.at[g, s],
                device_id=(my ^ mask,),
                device_id_type=pl.DeviceIdType.MESH,
            )
            rdma.start()
            return rdma

        def rs_accum(g, s):
            mask = MASKS[(g + s) % 3]
            half = SIZES[s]
            lower = beta_of[mask] == 0
            cg, w = COLS[g], W[g]
            keep_lo = _aligned(
                jnp.where(lower, jnp.int32(0), jnp.int32(half)), half
            )
            src = x_ref if s == 0 else acc_ref
            acc_ref[pl.ds(0, half), pl.ds(cg, w)] = (
                src[pl.ds(keep_lo, half), pl.ds(cg, w)]
                + comms[s][:, pl.ds(cg, w)].astype(jnp.float32)
            )
            offs[g] = offs[g] + jnp.where(lower, jnp.int32(0), jnp.int32(half))

        rs_d = [[None] * 3 for _ in range(NG)]
        for g in range(NG):
            rs_d[g][0] = rs_send(g, 0)
        for s in (1, 2):
            for g in range(NG):
                rs_d[g][s - 1].wait_recv()
                rs_accum(g, s - 1)
                rs_d[g][s] = rs_send(g, s)

        ag_recv = [[None] * 3 for _ in range(NG)]
        ag_poff = [[None] * 3 for _ in range(NG)]
        ag_sd = [[None] * 3 for _ in range(NG)]

        def ag_start(g, a):
            mask = MASKS[(g + 2 - a) % 3]
            lower = beta_of[mask] == 0
            sz = SZA[a]
            cg, w = COLS[g], W[g]
            o = _aligned(offs[g], sz)
            send = pltpu.make_async_remote_copy(
                src_ref=out_ref.at[pl.ds(o, sz), pl.ds(cg, w)],
                dst_ref=out_ref.at[pl.ds(o, sz), pl.ds(cg, w)],
                send_sem=ag_send_sems.at[g, a],
                recv_sem=ag_sems.at[g, a],
                device_id=(my ^ mask,),
                device_id_type=pl.DeviceIdType.MESH,
            )
            send.start()
            ag_sd[g][a] = send
            p_off = _aligned(jnp.where(lower, o + sz, o - sz), sz)
            recv = pltpu.make_async_remote_copy(
                src_ref=out_ref.at[pl.ds(p_off, sz), pl.ds(cg, w)],
                dst_ref=out_ref.at[pl.ds(p_off, sz), pl.ds(cg, w)],
                send_sem=ag_send_sems.at[g, a],
                recv_sem=ag_sems.at[g, a],
                device_id=(my ^ mask,),
                device_id_type=pl.DeviceIdType.MESH,
            )
            ag_recv[g][a] = recv
            ag_poff[g][a] = p_off

        for g in range(NG):
            rs_d[g][2].wait_recv()
            cg, w = COLS[g], W[g]
            s_val = (
                acc_ref[pl.ds(0, 512), pl.ds(cg, w)]
                + comms[2][:, pl.ds(cg, w)].astype(jnp.float32)
            )
            relu = jnp.maximum(s_val, 0.0)
            y = jnp.tanh(s_val) * s_val * s_val + relu * relu * relu
            out_ref[pl.ds(_aligned(offs[g], 512), 512), pl.ds(cg, w)] = (
                y.astype(bf16)
            )
            ag_start(g, 1)

        for g in range(NG):
            ag_recv[g][1].wait_recv()
            offs[g] = jnp.minimum(offs[g], ag_poff[g][1])
            ag_start(g, 2)
        for g in range(NG):
            ag_recv[g][2].wait_recv()
        for g in range(NG):
            for s in range(3):
                rs_d[g][s].wait_send()
            for a in (1, 2):
                ag_sd[g][a].wait_send()

    return pl.pallas_call(
        body,
        out_shape=jax.ShapeDtypeStruct((m, n), bf16),
        in_specs=[pl.BlockSpec(memory_space=pltpu.VMEM)],
        out_specs=pl.BlockSpec(memory_space=pltpu.VMEM),
        scratch_shapes=[
            pltpu.VMEM((m // 2, n), jnp.float32),
            pltpu.VMEM((m, n), bf16),
            pltpu.VMEM((m // 2, n), bf16),
            pltpu.VMEM((m // 4, n), bf16),
            pltpu.VMEM((m // 4, n), bf16),
            pltpu.SemaphoreType.DMA((NG, 3)),
            pltpu.SemaphoreType.DMA((NG, 3)),
            pltpu.SemaphoreType.DMA((NG, 3)),
            pltpu.SemaphoreType.DMA((NG, 3)),
        ],
        compiler_params=pltpu.CompilerParams(collective_id=0),
    )(t)
